# baseline (device time: 128080 ns/iter reference)
import jax
import jax.numpy as jnp
from jax import lax
from jax.experimental import pallas as pl
from jax.experimental.pallas import tpu as pltpu

N_Z = 4


def kernel(x):
    m_per, n = x.shape
    half = m_per // 2

    def body(x_ref, out_ref, z_send, z_recv, xf_send, xf_recv):
        my_x = lax.axis_index("x")
        my_y = lax.axis_index("y")
        my_z = lax.axis_index("z")
        xn = (1 - my_x, my_y, my_z)

        barrier = pltpu.get_barrier_semaphore()
        for d in range(1, N_Z):
            pl.semaphore_signal(
                barrier, inc=1,
                device_id=(my_x, my_y, (my_z + d) % N_Z),
                device_id_type=pl.DeviceIdType.MESH,
            )
        pl.semaphore_signal(
            barrier, inc=1, device_id=xn,
            device_id_type=pl.DeviceIdType.MESH,
        )
        pl.semaphore_wait(barrier, N_Z)

        def rdma(src, dst, ssem, rsem, dev):
            return pltpu.make_async_remote_copy(
                src_ref=src, dst_ref=dst, send_sem=ssem, recv_sem=rsem,
                device_id=dev, device_id_type=pl.DeviceIdType.MESH,
            )

        def out_half(z_origin, xh):
            return out_ref.at[pl.ds(z_origin * m_per + xh * half, half), :]

        for d in range(1, N_Z):
            rdma(
                x_ref.at[pl.ds(my_x * half, half), :],
                out_half(my_z, my_x),
                z_send.at[d - 1], z_recv.at[d - 1],
                (my_x, my_y, (my_z + d) % N_Z),
            ).start()

        out_ref[pl.ds(my_z * m_per, m_per), :] = x_ref[:, :]

        for d in range(1, N_Z):
            org = (my_z - d) % N_Z
            rdma(
                out_half(org, my_x), out_half(org, my_x),
                z_send.at[d - 1], z_recv.at[d - 1], xn,
            ).wait_recv()
            rdma(
                out_half(org, my_x), out_half(org, my_x),
                xf_send.at[d - 1], xf_recv.at[d - 1], xn,
            ).start()

        for d in range(1, N_Z):
            org = (my_z - d) % N_Z
            rdma(
                out_half(org, 1 - my_x), out_half(org, 1 - my_x),
                xf_send.at[d - 1], xf_recv.at[d - 1], xn,
            ).wait_recv()

        for d in range(1, N_Z):
            org = (my_z - d) % N_Z
            rdma(
                x_ref.at[pl.ds(my_x * half, half), :],
                out_half(my_z, my_x),
                z_send.at[d - 1], z_recv.at[d - 1],
                (my_x, my_y, (my_z + d) % N_Z),
            ).wait_send()
            rdma(
                out_half(org, my_x), out_half(org, my_x),
                xf_send.at[d - 1], xf_recv.at[d - 1], xn,
            ).wait_send()

    return pl.pallas_call(
        body,
        out_shape=jax.ShapeDtypeStruct((N_Z * m_per, n), x.dtype),
        in_specs=[pl.BlockSpec(memory_space=pltpu.VMEM)],
        out_specs=pl.BlockSpec(memory_space=pltpu.VMEM),
        scratch_shapes=[
            pltpu.SemaphoreType.DMA((N_Z - 1,)),
            pltpu.SemaphoreType.DMA((N_Z - 1,)),
            pltpu.SemaphoreType.DMA((N_Z - 1,)),
            pltpu.SemaphoreType.DMA((N_Z - 1,)),
        ],
        compiler_params=pltpu.CompilerParams(collective_id=0),
    )(x)
